# baseline (speedup 1.0000x reference)
import jax
import jax.numpy as jnp
from jax import lax
from jax.experimental import pallas as pl
from jax.experimental.pallas import tpu as pltpu
from jax.experimental.pallas import tpu_sc as plsc

N = 10000
E = 320000
D = 128

NC = 2
NS = 16
NW = NC * NS
L = 16

CHUNK = 128
EPT = E // NW
N_CH = 80
PAD_E = N_CH * CHUNK - EPT
N_PAD = 10112
RPT = N_PAD // NS

_MESH = dict(core_axis_name="c", subcore_axis_name="s",
             num_cores=NC, num_subcores=NS)
_SLAB = [(o, min(CHUNK, RPT - o)) for o in range(0, RPT, CHUNK)]


def _fill(ref, value, rows):
  v16 = jnp.full((L,), value, jnp.float32)

  def fb(i, carry):
    ref[i // (D // L), pl.ds((i % (D // L)) * L, L)] = v16
    return carry
  lax.fori_loop(0, rows * (D // L), fb, 0)


def _make_seg_sum():
  def body(table, srcp, dstp, acc_out, src_v, dst_v, rows0, acc_sh, sem_g):
    cid = lax.axis_index("c")
    sid = lax.axis_index("s")
    wid = cid * NS + sid
    base = sid * RPT

    _fill(rows0, 0.0, CHUNK)
    for off, sz in _SLAB:
      pltpu.sync_copy(rows0.at[pl.ds(0, sz)],
                      acc_sh.at[pl.ds(base + off, sz)])

    plsc.subcore_barrier()

    def edge_body(j, carry):
      pltpu.sync_copy(srcp.at[wid, pl.ds(j, 1)], src_v)
      pltpu.sync_copy(dstp.at[wid, pl.ds(j, 1)], dst_v)
      pltpu.async_copy(table.at[src_v.at[0]], rows0, sem_g).wait()
      pltpu.sync_copy(rows0, acc_sh.at[dst_v.at[0]], add=True)
      return carry
    lax.fori_loop(0, N_CH, edge_body, 0)

    plsc.subcore_barrier()
    for off, sz in _SLAB:
      pltpu.sync_copy(acc_sh.at[pl.ds(base + off, sz)],
                      rows0.at[pl.ds(0, sz)])
      pltpu.sync_copy(rows0.at[pl.ds(0, sz)],
                      acc_out.at[cid, pl.ds(base + off, sz)])

  return pl.kernel(
      body,
      out_type=jax.ShapeDtypeStruct((NC, N_PAD, D), jnp.float32),
      mesh=plsc.VectorSubcoreMesh(**_MESH),
      scratch_types=[
          pltpu.VMEM((1, CHUNK), jnp.int32),
          pltpu.VMEM((1, CHUNK), jnp.int32),
          pltpu.VMEM((CHUNK, D), jnp.float32),
          pltpu.VMEM_SHARED((N_PAD, D), jnp.float32),
          pltpu.SemaphoreType.DMA,
      ])


def _make_counts():
  def body(dstp, cnt_out, idx_v, rows_v, cnt_sh, sem_i):
    cid = lax.axis_index("c")
    sid = lax.axis_index("s")
    wid = cid * NS + sid
    base = sid * RPT

    _fill(rows_v, 0.0, CHUNK)
    for off, sz in _SLAB:
      pltpu.sync_copy(rows_v.at[pl.ds(0, sz)],
                      cnt_sh.at[pl.ds(base + off, sz)])
    _fill(rows_v, 1.0, CHUNK)
    pltpu.async_copy(dstp.at[wid, pl.ds(0, 1)], idx_v.at[0], sem_i)
    plsc.subcore_barrier()

    def edge_body(g, carry):
      for b in (0, 1):
        j = 2 * g + b
        pltpu.make_async_copy(dstp.at[wid, pl.ds(j, 1)],
                              idx_v.at[b], sem_i).wait()

        @pl.when(j + 1 < N_CH)
        def _():
          pltpu.async_copy(dstp.at[wid, pl.ds(j + 1, 1)],
                           idx_v.at[1 - b], sem_i)

        pltpu.sync_copy(rows_v, cnt_sh.at[idx_v.at[b, 0]], add=True)
      return carry
    lax.fori_loop(0, N_CH // 2, edge_body, 0)

    plsc.subcore_barrier()
    for off, sz in _SLAB:
      pltpu.sync_copy(cnt_sh.at[pl.ds(base + off, sz)],
                      rows_v.at[pl.ds(0, sz)])
      pltpu.sync_copy(rows_v.at[pl.ds(0, sz)],
                      cnt_out.at[cid, pl.ds(base + off, sz)])

  return pl.kernel(
      body,
      out_type=jax.ShapeDtypeStruct((NC, N_PAD, D), jnp.float32),
      mesh=plsc.VectorSubcoreMesh(**_MESH),
      scratch_types=[
          pltpu.VMEM((2, 1, CHUNK), jnp.int32),
          pltpu.VMEM((CHUNK, D), jnp.float32),
          pltpu.VMEM_SHARED((N_PAD, D), jnp.float32),
          pltpu.SemaphoreType.DMA,
      ])


_seg_sum = _make_seg_sum()
_counts = _make_counts()


def _tc_layer1(p_ref, cnt_ref, x_ref, wl_ref, b_ref, wr_ref, o_ref, inv_ref):
  s = p_ref[0] + p_ref[1]
  c = cnt_ref[0, :, 0] + cnt_ref[1, :, 0]
  inv = 1.0 / jnp.maximum(c, 1.0)
  mean = s * inv[:, None]
  o = (jnp.dot(mean, wl_ref[...], preferred_element_type=jnp.float32)
       + b_ref[...]
       + jnp.dot(x_ref[...], wr_ref[...], preferred_element_type=jnp.float32))
  o_ref[...] = jnp.maximum(o, 0.0)
  inv_ref[...] = jnp.broadcast_to(inv[:, None], inv_ref.shape)


def _tc_layer2(p_ref, inv_ref, x_ref, wl_ref, b_ref, wr_ref, o_ref):
  s = p_ref[0] + p_ref[1]
  mean = s * inv_ref[:, :1]
  o = (jnp.dot(mean, wl_ref[...], preferred_element_type=jnp.float32)
       + b_ref[...]
       + jnp.dot(x_ref[...], wr_ref[...], preferred_element_type=jnp.float32))
  nrm = jnp.sqrt(jnp.sum(o * o, axis=1, keepdims=True))
  o = o / jnp.maximum(nrm, 1e-12)
  o_ref[...] = jnp.maximum(o, 0.0)


BM = 128
_GRID = (N_PAD // BM,)

_tc1 = pl.pallas_call(
    _tc_layer1,
    grid=_GRID,
    in_specs=[
        pl.BlockSpec((NC, BM, D), lambda i: (0, i, 0)),
        pl.BlockSpec((NC, BM, D), lambda i: (0, i, 0)),
        pl.BlockSpec((BM, D), lambda i: (i, 0)),
        pl.BlockSpec((D, D), lambda i: (0, 0)),
        pl.BlockSpec((1, D), lambda i: (0, 0)),
        pl.BlockSpec((D, D), lambda i: (0, 0)),
    ],
    out_specs=(pl.BlockSpec((BM, D), lambda i: (i, 0)),
               pl.BlockSpec((BM, 8), lambda i: (i, 0))),
    out_shape=(jax.ShapeDtypeStruct((N_PAD, D), jnp.float32),
               jax.ShapeDtypeStruct((N_PAD, 8), jnp.float32)),
)

_tc2 = pl.pallas_call(
    _tc_layer2,
    grid=_GRID,
    in_specs=[
        pl.BlockSpec((NC, BM, D), lambda i: (0, i, 0)),
        pl.BlockSpec((BM, 8), lambda i: (i, 0)),
        pl.BlockSpec((BM, D), lambda i: (i, 0)),
        pl.BlockSpec((D, D), lambda i: (0, 0)),
        pl.BlockSpec((1, D), lambda i: (0, 0)),
        pl.BlockSpec((D, D), lambda i: (0, 0)),
    ],
    out_specs=pl.BlockSpec((BM, D), lambda i: (i, 0)),
    out_shape=jax.ShapeDtypeStruct((N_PAD, D), jnp.float32),
)


def kernel(matrix_nodes_features, edge_index, W1l, b1, W1r, W2l, b2, W2r):
  x = matrix_nodes_features.astype(jnp.float32)
  ei = edge_index.astype(jnp.int32)
  src = ei[0].reshape(NW, EPT)
  dst = ei[1].reshape(NW, EPT)
  srcp = jnp.concatenate(
      [src, jnp.zeros((NW, PAD_E), jnp.int32)], axis=1).reshape(NW, N_CH, CHUNK)
  dstp = jnp.concatenate(
      [dst, jnp.full((NW, PAD_E), N, jnp.int32)], axis=1).reshape(NW, N_CH, CHUNK)
  xp = jnp.concatenate([x, jnp.zeros((N_PAD - N, D), jnp.float32)], axis=0)
  b1r = b1.reshape(1, D).astype(jnp.float32)
  b2r = b2.reshape(1, D).astype(jnp.float32)

  cnt = _counts(dstp)
  p1 = _seg_sum(xp, srcp, dstp)
  h, inv8 = _tc1(p1, cnt, xp, W1l.astype(jnp.float32), b1r,
                 W1r.astype(jnp.float32))
  p2 = _seg_sum(h, srcp, dstp)
  out = _tc2(p2, inv8, h, W2l.astype(jnp.float32), b2r,
             W2r.astype(jnp.float32))
  return out[:N]


# speedup vs baseline: 1.0007x; 1.0007x over previous
import jax
import jax.numpy as jnp
from jax import lax
from jax.experimental import pallas as pl
from jax.experimental.pallas import tpu as pltpu
from jax.experimental.pallas import tpu_sc as plsc

N = 10000
E = 320000
D = 128

NC = 2
NS = 16
NW = NC * NS
L = 16

CHUNK = 128
EPT = E // NW
N_CH = 80
PAD_E = N_CH * CHUNK - EPT
N_PAD = 10240
RPT = N_PAD // NS

_MESH = dict(core_axis_name="c", subcore_axis_name="s",
             num_cores=NC, num_subcores=NS)
_SLAB = [(o, min(CHUNK, RPT - o)) for o in range(0, RPT, CHUNK)]


def _fill(ref, value, rows):
  v16 = jnp.full((L,), value, jnp.float32)

  def fb(i, carry):
    ref[i // (D // L), pl.ds((i % (D // L)) * L, L)] = v16
    return carry
  lax.fori_loop(0, rows * (D // L), fb, 0)


def _make_seg_sum():
  def body(table, srcp, dstp, acc_out, src_v, dst_v, rows0, acc_sh, sem_g):
    cid = lax.axis_index("c")
    sid = lax.axis_index("s")
    wid = cid * NS + sid
    base = sid * RPT

    _fill(rows0, 0.0, CHUNK)
    for off, sz in _SLAB:
      pltpu.sync_copy(rows0.at[pl.ds(0, sz)],
                      acc_sh.at[pl.ds(base + off, sz)])

    plsc.subcore_barrier()

    def edge_body(j, carry):
      pltpu.sync_copy(srcp.at[wid, pl.ds(j, 1)], src_v)
      pltpu.sync_copy(dstp.at[wid, pl.ds(j, 1)], dst_v)
      pltpu.async_copy(table.at[src_v.at[0]], rows0, sem_g).wait()
      pltpu.sync_copy(rows0, acc_sh.at[dst_v.at[0]], add=True)
      return carry
    lax.fori_loop(0, N_CH, edge_body, 0)

    plsc.subcore_barrier()
    for off, sz in _SLAB:
      pltpu.sync_copy(acc_sh.at[pl.ds(base + off, sz)],
                      rows0.at[pl.ds(0, sz)])
      pltpu.sync_copy(rows0.at[pl.ds(0, sz)],
                      acc_out.at[cid, pl.ds(base + off, sz)])

  return pl.kernel(
      body,
      out_type=jax.ShapeDtypeStruct((NC, N_PAD, D), jnp.float32),
      mesh=plsc.VectorSubcoreMesh(**_MESH),
      scratch_types=[
          pltpu.VMEM((1, CHUNK), jnp.int32),
          pltpu.VMEM((1, CHUNK), jnp.int32),
          pltpu.VMEM((CHUNK, D), jnp.float32),
          pltpu.VMEM_SHARED((N_PAD, D), jnp.float32),
          pltpu.SemaphoreType.DMA,
      ])


def _make_counts():
  def body(dstp, cnt_out, idx_v, rows_v, cnt_sh, sem_i):
    cid = lax.axis_index("c")
    sid = lax.axis_index("s")
    wid = cid * NS + sid
    base = sid * RPT

    _fill(rows_v, 0.0, CHUNK)
    for off, sz in _SLAB:
      pltpu.sync_copy(rows_v.at[pl.ds(0, sz)],
                      cnt_sh.at[pl.ds(base + off, sz)])
    _fill(rows_v, 1.0, CHUNK)
    pltpu.async_copy(dstp.at[wid, pl.ds(0, 1)], idx_v.at[0], sem_i)
    plsc.subcore_barrier()

    def edge_body(g, carry):
      for b in (0, 1):
        j = 2 * g + b
        pltpu.make_async_copy(dstp.at[wid, pl.ds(j, 1)],
                              idx_v.at[b], sem_i).wait()

        @pl.when(j + 1 < N_CH)
        def _():
          pltpu.async_copy(dstp.at[wid, pl.ds(j + 1, 1)],
                           idx_v.at[1 - b], sem_i)

        pltpu.sync_copy(rows_v, cnt_sh.at[idx_v.at[b, 0]], add=True)
      return carry
    lax.fori_loop(0, N_CH // 2, edge_body, 0)

    plsc.subcore_barrier()
    for off, sz in _SLAB:
      pltpu.sync_copy(cnt_sh.at[pl.ds(base + off, sz)],
                      rows_v.at[pl.ds(0, sz)])
      pltpu.sync_copy(rows_v.at[pl.ds(0, sz)],
                      cnt_out.at[cid, pl.ds(base + off, sz)])

  return pl.kernel(
      body,
      out_type=jax.ShapeDtypeStruct((NC, N_PAD, D), jnp.float32),
      mesh=plsc.VectorSubcoreMesh(**_MESH),
      scratch_types=[
          pltpu.VMEM((2, 1, CHUNK), jnp.int32),
          pltpu.VMEM((CHUNK, D), jnp.float32),
          pltpu.VMEM_SHARED((N_PAD, D), jnp.float32),
          pltpu.SemaphoreType.DMA,
      ])


_seg_sum = _make_seg_sum()
_counts = _make_counts()


def _tc_layer1(p_ref, cnt_ref, x_ref, wl_ref, b_ref, wr_ref, o_ref, inv_ref):
  s = p_ref[0] + p_ref[1]
  c = cnt_ref[0, :, 0] + cnt_ref[1, :, 0]
  inv = 1.0 / jnp.maximum(c, 1.0)
  mean = s * inv[:, None]
  o = (jnp.dot(mean, wl_ref[...], preferred_element_type=jnp.float32)
       + b_ref[...]
       + jnp.dot(x_ref[...], wr_ref[...], preferred_element_type=jnp.float32))
  o_ref[...] = jnp.maximum(o, 0.0)
  inv_ref[...] = jnp.broadcast_to(inv[:, None], inv_ref.shape)


def _tc_layer2(p_ref, inv_ref, x_ref, wl_ref, b_ref, wr_ref, o_ref):
  s = p_ref[0] + p_ref[1]
  mean = s * inv_ref[:, :1]
  o = (jnp.dot(mean, wl_ref[...], preferred_element_type=jnp.float32)
       + b_ref[...]
       + jnp.dot(x_ref[...], wr_ref[...], preferred_element_type=jnp.float32))
  nrm = jnp.sqrt(jnp.sum(o * o, axis=1, keepdims=True))
  o = o / jnp.maximum(nrm, 1e-12)
  o_ref[...] = jnp.maximum(o, 0.0)


BM = 128
_GRID = (N_PAD // BM,)

_tc1 = pl.pallas_call(
    _tc_layer1,
    grid=_GRID,
    in_specs=[
        pl.BlockSpec((NC, BM, D), lambda i: (0, i, 0)),
        pl.BlockSpec((NC, BM, D), lambda i: (0, i, 0)),
        pl.BlockSpec((BM, D), lambda i: (i, 0)),
        pl.BlockSpec((D, D), lambda i: (0, 0)),
        pl.BlockSpec((1, D), lambda i: (0, 0)),
        pl.BlockSpec((D, D), lambda i: (0, 0)),
    ],
    out_specs=(pl.BlockSpec((BM, D), lambda i: (i, 0)),
               pl.BlockSpec((BM, 8), lambda i: (i, 0))),
    out_shape=(jax.ShapeDtypeStruct((N_PAD, D), jnp.float32),
               jax.ShapeDtypeStruct((N_PAD, 8), jnp.float32)),
)

_tc2 = pl.pallas_call(
    _tc_layer2,
    grid=_GRID,
    in_specs=[
        pl.BlockSpec((NC, BM, D), lambda i: (0, i, 0)),
        pl.BlockSpec((BM, 8), lambda i: (i, 0)),
        pl.BlockSpec((BM, D), lambda i: (i, 0)),
        pl.BlockSpec((D, D), lambda i: (0, 0)),
        pl.BlockSpec((1, D), lambda i: (0, 0)),
        pl.BlockSpec((D, D), lambda i: (0, 0)),
    ],
    out_specs=pl.BlockSpec((BM, D), lambda i: (i, 0)),
    out_shape=jax.ShapeDtypeStruct((N_PAD, D), jnp.float32),
)


def kernel(matrix_nodes_features, edge_index, W1l, b1, W1r, W2l, b2, W2r):
  x = matrix_nodes_features.astype(jnp.float32)
  ei = edge_index.astype(jnp.int32)
  src = ei[0].reshape(NW, EPT)
  dst = ei[1].reshape(NW, EPT)
  srcp = jnp.concatenate(
      [src, jnp.zeros((NW, PAD_E), jnp.int32)], axis=1).reshape(NW, N_CH, CHUNK)
  pad_dst = N + (jnp.arange(PAD_E, dtype=jnp.int32) % (N_PAD - N))
  dstp = jnp.concatenate(
      [dst, jnp.broadcast_to(pad_dst, (NW, PAD_E))],
      axis=1).reshape(NW, N_CH, CHUNK)
  xp = jnp.concatenate([x, jnp.zeros((N_PAD - N, D), jnp.float32)], axis=0)
  b1r = b1.reshape(1, D).astype(jnp.float32)
  b2r = b2.reshape(1, D).astype(jnp.float32)

  cnt = _counts(dstp)
  p1 = _seg_sum(xp, srcp, dstp)
  h, inv8 = _tc1(p1, cnt, xp, W1l.astype(jnp.float32), b1r,
                 W1r.astype(jnp.float32))
  p2 = _seg_sum(h, srcp, dstp)
  out = _tc2(p2, inv8, h, W2l.astype(jnp.float32), b2r,
             W2r.astype(jnp.float32))
  return out[:N]


# speedup vs baseline: 1.2821x; 1.2813x over previous
import jax
import jax.numpy as jnp
from jax import lax
from jax.experimental import pallas as pl
from jax.experimental.pallas import tpu as pltpu
from jax.experimental.pallas import tpu_sc as plsc

N = 10000
E = 320000
D = 128

NC = 2
NS = 16
NW = NC * NS
L = 16

CHUNK = 128
EPT = E // NW
N_CH = 79
PAD_E = N_CH * CHUNK - EPT
N_PAD = 10112
RPT = N_PAD // NS

_MESH = dict(core_axis_name="c", subcore_axis_name="s",
             num_cores=NC, num_subcores=NS)
_SLAB = [(o, min(CHUNK, RPT - o)) for o in range(0, RPT, CHUNK)]


def _fill(ref, value, rows):
  v16 = jnp.full((L,), value, jnp.float32)

  def fb(i, carry):
    ref[i // (D // L), pl.ds((i % (D // L)) * L, L)] = v16
    return carry
  lax.fori_loop(0, rows * (D // L), fb, 0)


def _make_seg_sum():
  def body(table, srcp, dstp, acc_out, src_v, dst_v, rows0, acc_sh, sem_g):
    cid = lax.axis_index("c")
    sid = lax.axis_index("s")
    wid = cid * NS + sid
    base = sid * RPT

    _fill(rows0, 0.0, CHUNK)
    for off, sz in _SLAB:
      pltpu.sync_copy(rows0.at[pl.ds(0, sz)],
                      acc_sh.at[pl.ds(base + off, sz)])

    plsc.subcore_barrier()

    def edge_body(j, carry):
      pltpu.sync_copy(srcp.at[wid, pl.ds(j, 1)], src_v)
      pltpu.sync_copy(dstp.at[wid, pl.ds(j, 1)], dst_v)
      pltpu.async_copy(table.at[src_v.at[0]], rows0, sem_g).wait()
      pltpu.sync_copy(rows0, acc_sh.at[dst_v.at[0]], add=True)
      return carry
    lax.fori_loop(0, N_CH, edge_body, 0)

    plsc.subcore_barrier()
    for off, sz in _SLAB:
      pltpu.sync_copy(acc_sh.at[pl.ds(base + off, sz)],
                      rows0.at[pl.ds(0, sz)])
      pltpu.sync_copy(rows0.at[pl.ds(0, sz)],
                      acc_out.at[cid, pl.ds(base + off, sz)])

  return pl.kernel(
      body,
      out_type=jax.ShapeDtypeStruct((NC, N_PAD, D), jnp.float32),
      mesh=plsc.VectorSubcoreMesh(**_MESH),
      scratch_types=[
          pltpu.VMEM((1, CHUNK), jnp.int32),
          pltpu.VMEM((1, CHUNK), jnp.int32),
          pltpu.VMEM((CHUNK, D), jnp.float32),
          pltpu.VMEM_SHARED((N_PAD, D), jnp.float32),
          pltpu.SemaphoreType.DMA,
      ])


def _make_counts():
  def body(dstp, cnt_out, idx_v, rows_v, cnt_sh):
    cid = lax.axis_index("c")
    sid = lax.axis_index("s")
    wid = cid * NS + sid
    base = sid * RPT

    _fill(rows_v, 0.0, CHUNK)
    for off, sz in _SLAB:
      pltpu.sync_copy(rows_v.at[pl.ds(0, sz)],
                      cnt_sh.at[pl.ds(base + off, sz)])
    _fill(rows_v, 1.0, CHUNK)
    plsc.subcore_barrier()

    def edge_body(j, carry):
      pltpu.sync_copy(dstp.at[wid, pl.ds(j, 1)], idx_v)
      pltpu.sync_copy(rows_v, cnt_sh.at[idx_v.at[0]], add=True)
      return carry
    lax.fori_loop(0, N_CH, edge_body, 0)

    plsc.subcore_barrier()
    for off, sz in _SLAB:
      pltpu.sync_copy(cnt_sh.at[pl.ds(base + off, sz)],
                      rows_v.at[pl.ds(0, sz)])
      pltpu.sync_copy(rows_v.at[pl.ds(0, sz)],
                      cnt_out.at[cid, pl.ds(base + off, sz)])

  return pl.kernel(
      body,
      out_type=jax.ShapeDtypeStruct((NC, N_PAD, D), jnp.float32),
      mesh=plsc.VectorSubcoreMesh(**_MESH),
      scratch_types=[
          pltpu.VMEM((1, CHUNK), jnp.int32),
          pltpu.VMEM((CHUNK, D), jnp.float32),
          pltpu.VMEM_SHARED((N_PAD, D), jnp.float32),
      ])


_seg_sum = _make_seg_sum()
_counts = _make_counts()


def _tc_layer1(p_ref, cnt_ref, x_ref, wl_ref, b_ref, wr_ref, o_ref, inv_ref):
  s = p_ref[0] + p_ref[1]
  c = cnt_ref[0, :, 0] + cnt_ref[1, :, 0]
  inv = 1.0 / jnp.maximum(c, 1.0)
  mean = s * inv[:, None]
  o = (jnp.dot(mean, wl_ref[...], preferred_element_type=jnp.float32)
       + b_ref[...]
       + jnp.dot(x_ref[...], wr_ref[...], preferred_element_type=jnp.float32))
  o_ref[...] = jnp.maximum(o, 0.0)
  inv_ref[...] = jnp.broadcast_to(inv[:, None], inv_ref.shape)


def _tc_layer2(p_ref, inv_ref, x_ref, wl_ref, b_ref, wr_ref, o_ref):
  s = p_ref[0] + p_ref[1]
  mean = s * inv_ref[:, :1]
  o = (jnp.dot(mean, wl_ref[...], preferred_element_type=jnp.float32)
       + b_ref[...]
       + jnp.dot(x_ref[...], wr_ref[...], preferred_element_type=jnp.float32))
  nrm = jnp.sqrt(jnp.sum(o * o, axis=1, keepdims=True))
  o = o / jnp.maximum(nrm, 1e-12)
  o_ref[...] = jnp.maximum(o, 0.0)


BM = 128
_GRID = (N_PAD // BM,)

_tc1 = pl.pallas_call(
    _tc_layer1,
    grid=_GRID,
    in_specs=[
        pl.BlockSpec((NC, BM, D), lambda i: (0, i, 0)),
        pl.BlockSpec((NC, BM, D), lambda i: (0, i, 0)),
        pl.BlockSpec((BM, D), lambda i: (i, 0)),
        pl.BlockSpec((D, D), lambda i: (0, 0)),
        pl.BlockSpec((1, D), lambda i: (0, 0)),
        pl.BlockSpec((D, D), lambda i: (0, 0)),
    ],
    out_specs=(pl.BlockSpec((BM, D), lambda i: (i, 0)),
               pl.BlockSpec((BM, 8), lambda i: (i, 0))),
    out_shape=(jax.ShapeDtypeStruct((N_PAD, D), jnp.float32),
               jax.ShapeDtypeStruct((N_PAD, 8), jnp.float32)),
)

_tc2 = pl.pallas_call(
    _tc_layer2,
    grid=_GRID,
    in_specs=[
        pl.BlockSpec((NC, BM, D), lambda i: (0, i, 0)),
        pl.BlockSpec((BM, 8), lambda i: (i, 0)),
        pl.BlockSpec((BM, D), lambda i: (i, 0)),
        pl.BlockSpec((D, D), lambda i: (0, 0)),
        pl.BlockSpec((1, D), lambda i: (0, 0)),
        pl.BlockSpec((D, D), lambda i: (0, 0)),
    ],
    out_specs=pl.BlockSpec((BM, D), lambda i: (i, 0)),
    out_shape=jax.ShapeDtypeStruct((N_PAD, D), jnp.float32),
)


def kernel(matrix_nodes_features, edge_index, W1l, b1, W1r, W2l, b2, W2r):
  x = matrix_nodes_features.astype(jnp.float32)
  ei = edge_index.astype(jnp.int32)
  src = ei[0].reshape(NW, EPT)
  dst = ei[1].reshape(NW, EPT)
  srcp = jnp.concatenate(
      [src, jnp.zeros((NW, PAD_E), jnp.int32)], axis=1).reshape(NW, N_CH, CHUNK)
  pad_dst = N + (jnp.arange(PAD_E, dtype=jnp.int32) % (N_PAD - N))
  dstp = jnp.concatenate(
      [dst, jnp.broadcast_to(pad_dst, (NW, PAD_E))],
      axis=1).reshape(NW, N_CH, CHUNK)
  xp = jnp.concatenate([x, jnp.zeros((N_PAD - N, D), jnp.float32)], axis=0)
  b1r = b1.reshape(1, D).astype(jnp.float32)
  b2r = b2.reshape(1, D).astype(jnp.float32)

  cnt = _counts(dstp)
  p1 = _seg_sum(xp, srcp, dstp)
  h, inv8 = _tc1(p1, cnt, xp, W1l.astype(jnp.float32), b1r,
                 W1r.astype(jnp.float32))
  p2 = _seg_sum(h, srcp, dstp)
  out = _tc2(p2, inv8, h, W2l.astype(jnp.float32), b2r,
             W2r.astype(jnp.float32))
  return out[:N]


# speedup vs baseline: 1.3228x; 1.0318x over previous
import jax
import jax.numpy as jnp
from jax import lax
from jax.experimental import pallas as pl
from jax.experimental.pallas import tpu as pltpu
from jax.experimental.pallas import tpu_sc as plsc

N = 10000
E = 320000
D = 128

NC = 2
NS = 16
NW = NC * NS
L = 16

CHUNK = 128
EPT = E // NW
N_CH = 79
PAD_E = N_CH * CHUNK - EPT
N_PAD = 10112
RPT = N_PAD // NS

_MESH = dict(core_axis_name="c", subcore_axis_name="s",
             num_cores=NC, num_subcores=NS)
_SLAB = [(o, min(CHUNK, RPT - o)) for o in range(0, RPT, CHUNK)]


def _fill(ref, value, rows):
  v16 = jnp.full((L,), value, jnp.float32)

  def fb(i, carry):
    ref[i // (D // L), pl.ds((i % (D // L)) * L, L)] = v16
    return carry
  lax.fori_loop(0, rows * (D // L), fb, 0)


def _make_seg_sum():
  def body(table, srcp, dstp, acc_out, src_v, dst_v, rows0, acc_sh, sem_g):
    cid = lax.axis_index("c")
    sid = lax.axis_index("s")
    wid = cid * NS + sid
    base = sid * RPT

    _fill(rows0, 0.0, CHUNK)
    for off, sz in _SLAB:
      pltpu.sync_copy(rows0.at[pl.ds(0, sz)],
                      acc_sh.at[pl.ds(base + off, sz)])

    plsc.subcore_barrier()

    def edge_body(j, carry):
      pltpu.sync_copy(srcp.at[wid, pl.ds(j, 1)], src_v)
      pltpu.sync_copy(dstp.at[wid, pl.ds(j, 1)], dst_v)
      pltpu.async_copy(table.at[src_v.at[0]], rows0, sem_g).wait()
      pltpu.sync_copy(rows0, acc_sh.at[dst_v.at[0]], add=True)
      return carry
    lax.fori_loop(0, N_CH, edge_body, 0)

    plsc.subcore_barrier()
    for off, sz in _SLAB:
      pltpu.sync_copy(acc_sh.at[pl.ds(base + off, sz)],
                      rows0.at[pl.ds(0, sz)])
      pltpu.sync_copy(rows0.at[pl.ds(0, sz)],
                      acc_out.at[cid, pl.ds(base + off, sz)])

  return pl.kernel(
      body,
      out_type=jax.ShapeDtypeStruct((NC, N_PAD, D), jnp.float32),
      mesh=plsc.VectorSubcoreMesh(**_MESH),
      scratch_types=[
          pltpu.VMEM((1, CHUNK), jnp.int32),
          pltpu.VMEM((1, CHUNK), jnp.int32),
          pltpu.VMEM((CHUNK, D), jnp.float32),
          pltpu.VMEM_SHARED((N_PAD, D), jnp.float32),
          pltpu.SemaphoreType.DMA,
      ])


def _make_counts():
  def body(dstp, cnt_out, idx_v, rows_v, cnt_sh, sem_i):
    cid = lax.axis_index("c")
    sid = lax.axis_index("s")
    wid = cid * NS + sid
    base = sid * RPT

    _fill(rows_v, 0.0, CHUNK)
    for off, sz in _SLAB:
      pltpu.sync_copy(rows_v.at[pl.ds(0, sz)],
                      cnt_sh.at[pl.ds(base + off, sz)])
    _fill(rows_v, 1.0, CHUNK)
    pltpu.async_copy(dstp.at[wid, pl.ds(0, 1)], idx_v.at[0], sem_i)
    plsc.subcore_barrier()

    def edge_body(g, carry):
      for b in (0, 1):
        j = 2 * g + b
        pltpu.make_async_copy(dstp.at[wid, pl.ds(j, 1)],
                              idx_v.at[b], sem_i).wait()
        pltpu.async_copy(dstp.at[wid, pl.ds(j + 1, 1)],
                         idx_v.at[1 - b], sem_i)
        pltpu.sync_copy(rows_v, cnt_sh.at[idx_v.at[b, 0]], add=True)
      return carry
    lax.fori_loop(0, N_CH // 2, edge_body, 0)
    pltpu.make_async_copy(dstp.at[wid, pl.ds(N_CH - 1, 1)],
                          idx_v.at[0], sem_i).wait()
    pltpu.sync_copy(rows_v, cnt_sh.at[idx_v.at[0, 0]], add=True)

    plsc.subcore_barrier()
    for off, sz in _SLAB:
      pltpu.sync_copy(cnt_sh.at[pl.ds(base + off, sz)],
                      rows_v.at[pl.ds(0, sz)])
      pltpu.sync_copy(rows_v.at[pl.ds(0, sz)],
                      cnt_out.at[cid, pl.ds(base + off, sz)])

  return pl.kernel(
      body,
      out_type=jax.ShapeDtypeStruct((NC, N_PAD, D), jnp.float32),
      mesh=plsc.VectorSubcoreMesh(**_MESH),
      scratch_types=[
          pltpu.VMEM((2, 1, CHUNK), jnp.int32),
          pltpu.VMEM((CHUNK, D), jnp.float32),
          pltpu.VMEM_SHARED((N_PAD, D), jnp.float32),
          pltpu.SemaphoreType.DMA,
      ])


_seg_sum = _make_seg_sum()
_counts = _make_counts()


def _tc_layer1(p_ref, cnt_ref, x_ref, wl_ref, b_ref, wr_ref, o_ref, inv_ref):
  s = p_ref[0] + p_ref[1]
  c = cnt_ref[0, :, 0] + cnt_ref[1, :, 0]
  inv = 1.0 / jnp.maximum(c, 1.0)
  mean = s * inv[:, None]
  o = (jnp.dot(mean, wl_ref[...], preferred_element_type=jnp.float32)
       + b_ref[...]
       + jnp.dot(x_ref[...], wr_ref[...], preferred_element_type=jnp.float32))
  o_ref[...] = jnp.maximum(o, 0.0)
  inv_ref[...] = jnp.broadcast_to(inv[:, None], inv_ref.shape)


def _tc_layer2(p_ref, inv_ref, x_ref, wl_ref, b_ref, wr_ref, o_ref):
  s = p_ref[0] + p_ref[1]
  mean = s * inv_ref[:, :1]
  o = (jnp.dot(mean, wl_ref[...], preferred_element_type=jnp.float32)
       + b_ref[...]
       + jnp.dot(x_ref[...], wr_ref[...], preferred_element_type=jnp.float32))
  nrm = jnp.sqrt(jnp.sum(o * o, axis=1, keepdims=True))
  o = o / jnp.maximum(nrm, 1e-12)
  o_ref[...] = jnp.maximum(o, 0.0)


BM = 128
_GRID = (N_PAD // BM,)

_tc1 = pl.pallas_call(
    _tc_layer1,
    grid=_GRID,
    in_specs=[
        pl.BlockSpec((NC, BM, D), lambda i: (0, i, 0)),
        pl.BlockSpec((NC, BM, D), lambda i: (0, i, 0)),
        pl.BlockSpec((BM, D), lambda i: (i, 0)),
        pl.BlockSpec((D, D), lambda i: (0, 0)),
        pl.BlockSpec((1, D), lambda i: (0, 0)),
        pl.BlockSpec((D, D), lambda i: (0, 0)),
    ],
    out_specs=(pl.BlockSpec((BM, D), lambda i: (i, 0)),
               pl.BlockSpec((BM, 8), lambda i: (i, 0))),
    out_shape=(jax.ShapeDtypeStruct((N_PAD, D), jnp.float32),
               jax.ShapeDtypeStruct((N_PAD, 8), jnp.float32)),
)

_tc2 = pl.pallas_call(
    _tc_layer2,
    grid=_GRID,
    in_specs=[
        pl.BlockSpec((NC, BM, D), lambda i: (0, i, 0)),
        pl.BlockSpec((BM, 8), lambda i: (i, 0)),
        pl.BlockSpec((BM, D), lambda i: (i, 0)),
        pl.BlockSpec((D, D), lambda i: (0, 0)),
        pl.BlockSpec((1, D), lambda i: (0, 0)),
        pl.BlockSpec((D, D), lambda i: (0, 0)),
    ],
    out_specs=pl.BlockSpec((BM, D), lambda i: (i, 0)),
    out_shape=jax.ShapeDtypeStruct((N_PAD, D), jnp.float32),
)


def kernel(matrix_nodes_features, edge_index, W1l, b1, W1r, W2l, b2, W2r):
  x = matrix_nodes_features.astype(jnp.float32)
  ei = edge_index.astype(jnp.int32)
  src = ei[0].reshape(NW, EPT)
  dst = ei[1].reshape(NW, EPT)
  srcp = jnp.concatenate(
      [src, jnp.zeros((NW, PAD_E), jnp.int32)], axis=1).reshape(NW, N_CH, CHUNK)
  pad_dst = N + (jnp.arange(PAD_E, dtype=jnp.int32) % (N_PAD - N))
  dstp = jnp.concatenate(
      [dst, jnp.broadcast_to(pad_dst, (NW, PAD_E))],
      axis=1).reshape(NW, N_CH, CHUNK)
  xp = jnp.concatenate([x, jnp.zeros((N_PAD - N, D), jnp.float32)], axis=0)
  b1r = b1.reshape(1, D).astype(jnp.float32)
  b2r = b2.reshape(1, D).astype(jnp.float32)

  cnt = _counts(dstp)
  p1 = _seg_sum(xp, srcp, dstp)
  h, inv8 = _tc1(p1, cnt, xp, W1l.astype(jnp.float32), b1r,
                 W1r.astype(jnp.float32))
  p2 = _seg_sum(h, srcp, dstp)
  out = _tc2(p2, inv8, h, W2l.astype(jnp.float32), b2r,
             W2r.astype(jnp.float32))
  return out[:N]


# speedup vs baseline: 1.5343x; 1.1599x over previous
import jax
import jax.numpy as jnp
from jax import lax
from jax.experimental import pallas as pl
from jax.experimental.pallas import tpu as pltpu
from jax.experimental.pallas import tpu_sc as plsc

N = 10000
E = 320000
D = 128

NC = 2
NS = 16
NW = NC * NS
L = 16

CHUNK = 128
EPT = E // NW
N_CH = 79
PAD_E = N_CH * CHUNK - EPT
N_PAD = 10112
RPT = N_PAD // NS

_MESH = dict(core_axis_name="c", subcore_axis_name="s",
             num_cores=NC, num_subcores=NS)
_SLAB = [(o, min(CHUNK, RPT - o)) for o in range(0, RPT, CHUNK)]


def _fill(ref, value, rows):
  v16 = jnp.full((L,), value, jnp.float32)

  def fb(i, carry):
    ref[i // (D // L), pl.ds((i % (D // L)) * L, L)] = v16
    return carry
  lax.fori_loop(0, rows * (D // L), fb, 0)


def _make_seg_sum():
  def body(table, srcp, dstp, acc_out, src_v, dst_v, rows0, acc_sh, sem_g,
           sem_i):
    cid = lax.axis_index("c")
    sid = lax.axis_index("s")
    wid = cid * NS + sid
    base = sid * RPT

    _fill(rows0, 0.0, CHUNK)
    for off, sz in _SLAB:
      pltpu.sync_copy(rows0.at[pl.ds(0, sz)],
                      acc_sh.at[pl.ds(base + off, sz)])

    pltpu.async_copy(srcp.at[wid, pl.ds(0, 1)], src_v.at[0], sem_i)
    pltpu.async_copy(dstp.at[wid, pl.ds(0, 1)], dst_v.at[0], sem_i)
    plsc.subcore_barrier()

    def edge_body(g, carry):
      for b in (0, 1):
        j = 2 * g + b
        pltpu.make_async_copy(srcp.at[wid, pl.ds(j, 1)],
                              src_v.at[b], sem_i).wait()
        pltpu.make_async_copy(dstp.at[wid, pl.ds(j, 1)],
                              dst_v.at[b], sem_i).wait()
        pltpu.async_copy(srcp.at[wid, pl.ds(j + 1, 1)], src_v.at[1 - b], sem_i)
        pltpu.async_copy(dstp.at[wid, pl.ds(j + 1, 1)], dst_v.at[1 - b], sem_i)
        pltpu.async_copy(table.at[src_v.at[b, 0]], rows0, sem_g).wait()
        pltpu.sync_copy(rows0, acc_sh.at[dst_v.at[b, 0]], add=True)
      return carry
    lax.fori_loop(0, N_CH // 2, edge_body, 0)
    pltpu.make_async_copy(srcp.at[wid, pl.ds(N_CH - 1, 1)],
                          src_v.at[0], sem_i).wait()
    pltpu.make_async_copy(dstp.at[wid, pl.ds(N_CH - 1, 1)],
                          dst_v.at[0], sem_i).wait()
    pltpu.async_copy(table.at[src_v.at[0, 0]], rows0, sem_g).wait()
    pltpu.sync_copy(rows0, acc_sh.at[dst_v.at[0, 0]], add=True)

    plsc.subcore_barrier()
    for off, sz in _SLAB:
      pltpu.sync_copy(acc_sh.at[pl.ds(base + off, sz)],
                      rows0.at[pl.ds(0, sz)])
      pltpu.sync_copy(rows0.at[pl.ds(0, sz)],
                      acc_out.at[cid, pl.ds(base + off, sz)])

  return pl.kernel(
      body,
      out_type=jax.ShapeDtypeStruct((NC, N_PAD, D), jnp.float32),
      mesh=plsc.VectorSubcoreMesh(**_MESH),
      scratch_types=[
          pltpu.VMEM((2, 1, CHUNK), jnp.int32),
          pltpu.VMEM((2, 1, CHUNK), jnp.int32),
          pltpu.VMEM((CHUNK, D), jnp.float32),
          pltpu.VMEM_SHARED((N_PAD, D), jnp.float32),
          pltpu.SemaphoreType.DMA,
          pltpu.SemaphoreType.DMA,
      ])


def _make_counts():
  def body(dstp, cnt_out, idx_v, rows_v, cnt_sh, sem_i):
    cid = lax.axis_index("c")
    sid = lax.axis_index("s")
    wid = cid * NS + sid
    base = sid * RPT

    _fill(rows_v, 0.0, CHUNK)
    for off, sz in _SLAB:
      pltpu.sync_copy(rows_v.at[pl.ds(0, sz)],
                      cnt_sh.at[pl.ds(base + off, sz)])
    _fill(rows_v, 1.0, CHUNK)
    pltpu.async_copy(dstp.at[wid, pl.ds(0, 1)], idx_v.at[0], sem_i)
    plsc.subcore_barrier()

    def edge_body(g, carry):
      for b in (0, 1):
        j = 2 * g + b
        pltpu.make_async_copy(dstp.at[wid, pl.ds(j, 1)],
                              idx_v.at[b], sem_i).wait()
        pltpu.async_copy(dstp.at[wid, pl.ds(j + 1, 1)],
                         idx_v.at[1 - b], sem_i)
        pltpu.sync_copy(rows_v, cnt_sh.at[idx_v.at[b, 0]], add=True)
      return carry
    lax.fori_loop(0, N_CH // 2, edge_body, 0)
    pltpu.make_async_copy(dstp.at[wid, pl.ds(N_CH - 1, 1)],
                          idx_v.at[0], sem_i).wait()
    pltpu.sync_copy(rows_v, cnt_sh.at[idx_v.at[0, 0]], add=True)

    plsc.subcore_barrier()
    for off, sz in _SLAB:
      pltpu.sync_copy(cnt_sh.at[pl.ds(base + off, sz)],
                      rows_v.at[pl.ds(0, sz)])
      pltpu.sync_copy(rows_v.at[pl.ds(0, sz)],
                      cnt_out.at[cid, pl.ds(base + off, sz)])

  return pl.kernel(
      body,
      out_type=jax.ShapeDtypeStruct((NC, N_PAD, D), jnp.float32),
      mesh=plsc.VectorSubcoreMesh(**_MESH),
      scratch_types=[
          pltpu.VMEM((2, 1, CHUNK), jnp.int32),
          pltpu.VMEM((CHUNK, D), jnp.float32),
          pltpu.VMEM_SHARED((N_PAD, D), jnp.float32),
          pltpu.SemaphoreType.DMA,
      ])


_seg_sum = _make_seg_sum()
_counts = _make_counts()


def _tc_layer1(p_ref, cnt_ref, x_ref, wl_ref, b_ref, wr_ref, o_ref, inv_ref):
  s = p_ref[0] + p_ref[1]
  c = cnt_ref[0, :, 0] + cnt_ref[1, :, 0]
  inv = 1.0 / jnp.maximum(c, 1.0)
  mean = s * inv[:, None]
  o = (jnp.dot(mean, wl_ref[...], preferred_element_type=jnp.float32)
       + b_ref[...]
       + jnp.dot(x_ref[...], wr_ref[...], preferred_element_type=jnp.float32))
  o_ref[...] = jnp.maximum(o, 0.0)
  inv_ref[...] = jnp.broadcast_to(inv[:, None], inv_ref.shape)


def _tc_layer2(p_ref, inv_ref, x_ref, wl_ref, b_ref, wr_ref, o_ref):
  s = p_ref[0] + p_ref[1]
  mean = s * inv_ref[:, :1]
  o = (jnp.dot(mean, wl_ref[...], preferred_element_type=jnp.float32)
       + b_ref[...]
       + jnp.dot(x_ref[...], wr_ref[...], preferred_element_type=jnp.float32))
  nrm = jnp.sqrt(jnp.sum(o * o, axis=1, keepdims=True))
  o = o / jnp.maximum(nrm, 1e-12)
  o_ref[...] = jnp.maximum(o, 0.0)


BM = 128
_GRID = (N_PAD // BM,)

_tc1 = pl.pallas_call(
    _tc_layer1,
    grid=_GRID,
    in_specs=[
        pl.BlockSpec((NC, BM, D), lambda i: (0, i, 0)),
        pl.BlockSpec((NC, BM, D), lambda i: (0, i, 0)),
        pl.BlockSpec((BM, D), lambda i: (i, 0)),
        pl.BlockSpec((D, D), lambda i: (0, 0)),
        pl.BlockSpec((1, D), lambda i: (0, 0)),
        pl.BlockSpec((D, D), lambda i: (0, 0)),
    ],
    out_specs=(pl.BlockSpec((BM, D), lambda i: (i, 0)),
               pl.BlockSpec((BM, 8), lambda i: (i, 0))),
    out_shape=(jax.ShapeDtypeStruct((N_PAD, D), jnp.float32),
               jax.ShapeDtypeStruct((N_PAD, 8), jnp.float32)),
)

_tc2 = pl.pallas_call(
    _tc_layer2,
    grid=_GRID,
    in_specs=[
        pl.BlockSpec((NC, BM, D), lambda i: (0, i, 0)),
        pl.BlockSpec((BM, 8), lambda i: (i, 0)),
        pl.BlockSpec((BM, D), lambda i: (i, 0)),
        pl.BlockSpec((D, D), lambda i: (0, 0)),
        pl.BlockSpec((1, D), lambda i: (0, 0)),
        pl.BlockSpec((D, D), lambda i: (0, 0)),
    ],
    out_specs=pl.BlockSpec((BM, D), lambda i: (i, 0)),
    out_shape=jax.ShapeDtypeStruct((N_PAD, D), jnp.float32),
)


def kernel(matrix_nodes_features, edge_index, W1l, b1, W1r, W2l, b2, W2r):
  x = matrix_nodes_features.astype(jnp.float32)
  ei = edge_index.astype(jnp.int32)
  src = ei[0].reshape(NW, EPT)
  dst = ei[1].reshape(NW, EPT)
  srcp = jnp.concatenate(
      [src, jnp.zeros((NW, PAD_E), jnp.int32)], axis=1).reshape(NW, N_CH, CHUNK)
  pad_dst = N + (jnp.arange(PAD_E, dtype=jnp.int32) % (N_PAD - N))
  dstp = jnp.concatenate(
      [dst, jnp.broadcast_to(pad_dst, (NW, PAD_E))],
      axis=1).reshape(NW, N_CH, CHUNK)
  xp = jnp.concatenate([x, jnp.zeros((N_PAD - N, D), jnp.float32)], axis=0)
  b1r = b1.reshape(1, D).astype(jnp.float32)
  b2r = b2.reshape(1, D).astype(jnp.float32)

  cnt = _counts(dstp)
  p1 = _seg_sum(xp, srcp, dstp)
  h, inv8 = _tc1(p1, cnt, xp, W1l.astype(jnp.float32), b1r,
                 W1r.astype(jnp.float32))
  p2 = _seg_sum(h, srcp, dstp)
  out = _tc2(p2, inv8, h, W2l.astype(jnp.float32), b2r,
             W2r.astype(jnp.float32))
  return out[:N]


# speedup vs baseline: 2.6913x; 1.7541x over previous
import jax
import jax.numpy as jnp
from jax import lax
from jax.experimental import pallas as pl
from jax.experimental.pallas import tpu as pltpu
from jax.experimental.pallas import tpu_sc as plsc

N = 10000
E = 320000
D = 128

NC = 2
NS = 16
NW = NC * NS
L = 16

CHUNK = 128
EPT = E // NW
N_CH = 79
PAD_E = N_CH * CHUNK - EPT
N_PAD = 10112
RPT = N_PAD // NS

_MESH = dict(core_axis_name="c", subcore_axis_name="s",
             num_cores=NC, num_subcores=NS)
_SLAB = [(o, min(CHUNK, RPT - o)) for o in range(0, RPT, CHUNK)]


def _fill(ref, value, rows):
  v16 = jnp.full((L,), value, jnp.float32)

  def fb(i, carry):
    ref[i // (D // L), pl.ds((i % (D // L)) * L, L)] = v16
    return carry
  lax.fori_loop(0, rows * (D // L), fb, 0)


def _make_seg_sum():
  def body(table, srcp, dstp, acc_out, src_v, dst_v, rows0, rows1, acc_sh,
           sem_g, sem_i):
    cid = lax.axis_index("c")
    sid = lax.axis_index("s")
    wid = cid * NS + sid
    base = sid * RPT

    _fill(rows0, 0.0, CHUNK)
    for off, sz in _SLAB:
      pltpu.sync_copy(rows0.at[pl.ds(0, sz)],
                      acc_sh.at[pl.ds(base + off, sz)])

    rows = (rows0, rows1)
    pltpu.async_copy(srcp.at[wid, pl.ds(0, 1)], src_v.at[0], sem_i)
    pltpu.async_copy(dstp.at[wid, pl.ds(0, 1)], dst_v.at[0], sem_i)
    pltpu.async_copy(srcp.at[wid, pl.ds(1, 1)], src_v.at[1], sem_i)
    pltpu.async_copy(dstp.at[wid, pl.ds(1, 1)], dst_v.at[1], sem_i)
    pltpu.make_async_copy(srcp.at[wid, pl.ds(0, 1)], src_v.at[0], sem_i).wait()
    pltpu.async_copy(table.at[src_v.at[0, 0]], rows0, sem_g)
    plsc.subcore_barrier()

    def edge_body(g, carry):
      for b in (0, 1):
        j = 2 * g + b
        pltpu.make_async_copy(srcp.at[wid, pl.ds(j + 1, 1)],
                              src_v.at[1 - b], sem_i).wait()
        pltpu.async_copy(table.at[src_v.at[1 - b, 0]], rows[1 - b], sem_g)
        pltpu.make_async_copy(table.at[src_v.at[b, 0]], rows[b], sem_g).wait()
        pltpu.make_async_copy(dstp.at[wid, pl.ds(j, 1)],
                              dst_v.at[b], sem_i).wait()
        pltpu.sync_copy(rows[b], acc_sh.at[dst_v.at[b, 0]], add=True)
        pltpu.async_copy(srcp.at[wid, pl.ds(j + 2, 1)], src_v.at[b], sem_i)
        pltpu.async_copy(dstp.at[wid, pl.ds(j + 2, 1)], dst_v.at[b], sem_i)
      return carry
    lax.fori_loop(0, N_CH // 2, edge_body, 0)
    pltpu.make_async_copy(table.at[src_v.at[0, 0]], rows0, sem_g).wait()
    pltpu.make_async_copy(dstp.at[wid, pl.ds(N_CH - 1, 1)],
                          dst_v.at[0], sem_i).wait()
    pltpu.sync_copy(rows0, acc_sh.at[dst_v.at[0, 0]], add=True)
    pltpu.make_async_copy(srcp.at[wid, pl.ds(N_CH, 1)],
                          src_v.at[1], sem_i).wait()
    pltpu.make_async_copy(dstp.at[wid, pl.ds(N_CH, 1)],
                          dst_v.at[1], sem_i).wait()

    plsc.subcore_barrier()
    for off, sz in _SLAB:
      pltpu.sync_copy(acc_sh.at[pl.ds(base + off, sz)],
                      rows0.at[pl.ds(0, sz)])
      pltpu.sync_copy(rows0.at[pl.ds(0, sz)],
                      acc_out.at[cid, pl.ds(base + off, sz)])

  return pl.kernel(
      body,
      out_type=jax.ShapeDtypeStruct((NC, N_PAD, D), jnp.float32),
      mesh=plsc.VectorSubcoreMesh(**_MESH),
      scratch_types=[
          pltpu.VMEM((2, 1, CHUNK), jnp.int32),
          pltpu.VMEM((2, 1, CHUNK), jnp.int32),
          pltpu.VMEM((CHUNK, D), jnp.float32),
          pltpu.VMEM((CHUNK, D), jnp.float32),
          pltpu.VMEM_SHARED((N_PAD, D), jnp.float32),
          pltpu.SemaphoreType.DMA,
          pltpu.SemaphoreType.DMA,
      ])


def _make_counts():
  def body(dstp, cnt_out, idx_v, rows_v, cnt_sh, sem_i):
    cid = lax.axis_index("c")
    sid = lax.axis_index("s")
    wid = cid * NS + sid
    base = sid * RPT

    _fill(rows_v, 0.0, CHUNK)
    for off, sz in _SLAB:
      pltpu.sync_copy(rows_v.at[pl.ds(0, sz)],
                      cnt_sh.at[pl.ds(base + off, sz)])
    _fill(rows_v, 1.0, CHUNK)
    pltpu.async_copy(dstp.at[wid, pl.ds(0, 1)], idx_v.at[0], sem_i)
    plsc.subcore_barrier()

    def edge_body(g, carry):
      for b in (0, 1):
        j = 2 * g + b
        pltpu.make_async_copy(dstp.at[wid, pl.ds(j, 1)],
                              idx_v.at[b], sem_i).wait()
        pltpu.async_copy(dstp.at[wid, pl.ds(j + 1, 1)],
                         idx_v.at[1 - b], sem_i)
        pltpu.sync_copy(rows_v, cnt_sh.at[idx_v.at[b, 0]], add=True)
      return carry
    lax.fori_loop(0, N_CH // 2, edge_body, 0)
    pltpu.make_async_copy(dstp.at[wid, pl.ds(N_CH - 1, 1)],
                          idx_v.at[0], sem_i).wait()
    pltpu.sync_copy(rows_v, cnt_sh.at[idx_v.at[0, 0]], add=True)

    plsc.subcore_barrier()
    for off, sz in _SLAB:
      pltpu.sync_copy(cnt_sh.at[pl.ds(base + off, sz)],
                      rows_v.at[pl.ds(0, sz)])
      pltpu.sync_copy(rows_v.at[pl.ds(0, sz)],
                      cnt_out.at[cid, pl.ds(base + off, sz)])

  return pl.kernel(
      body,
      out_type=jax.ShapeDtypeStruct((NC, N_PAD, D), jnp.float32),
      mesh=plsc.VectorSubcoreMesh(**_MESH),
      scratch_types=[
          pltpu.VMEM((2, 1, CHUNK), jnp.int32),
          pltpu.VMEM((CHUNK, D), jnp.float32),
          pltpu.VMEM_SHARED((N_PAD, D), jnp.float32),
          pltpu.SemaphoreType.DMA,
      ])


_seg_sum = _make_seg_sum()
_counts = _make_counts()


def _tc_layer1(p_ref, cnt_ref, x_ref, wl_ref, b_ref, wr_ref, o_ref, inv_ref):
  s = p_ref[0] + p_ref[1]
  c = cnt_ref[0, :, 0] + cnt_ref[1, :, 0]
  inv = 1.0 / jnp.maximum(c, 1.0)
  mean = s * inv[:, None]
  o = (jnp.dot(mean, wl_ref[...], preferred_element_type=jnp.float32)
       + b_ref[...]
       + jnp.dot(x_ref[...], wr_ref[...], preferred_element_type=jnp.float32))
  o_ref[...] = jnp.maximum(o, 0.0)
  inv_ref[...] = jnp.broadcast_to(inv[:, None], inv_ref.shape)


def _tc_layer2(p_ref, inv_ref, x_ref, wl_ref, b_ref, wr_ref, o_ref):
  s = p_ref[0] + p_ref[1]
  mean = s * inv_ref[:, :1]
  o = (jnp.dot(mean, wl_ref[...], preferred_element_type=jnp.float32)
       + b_ref[...]
       + jnp.dot(x_ref[...], wr_ref[...], preferred_element_type=jnp.float32))
  nrm = jnp.sqrt(jnp.sum(o * o, axis=1, keepdims=True))
  o = o / jnp.maximum(nrm, 1e-12)
  o_ref[...] = jnp.maximum(o, 0.0)


BM = 128
_GRID = (N_PAD // BM,)

_tc1 = pl.pallas_call(
    _tc_layer1,
    grid=_GRID,
    in_specs=[
        pl.BlockSpec((NC, BM, D), lambda i: (0, i, 0)),
        pl.BlockSpec((NC, BM, D), lambda i: (0, i, 0)),
        pl.BlockSpec((BM, D), lambda i: (i, 0)),
        pl.BlockSpec((D, D), lambda i: (0, 0)),
        pl.BlockSpec((1, D), lambda i: (0, 0)),
        pl.BlockSpec((D, D), lambda i: (0, 0)),
    ],
    out_specs=(pl.BlockSpec((BM, D), lambda i: (i, 0)),
               pl.BlockSpec((BM, 8), lambda i: (i, 0))),
    out_shape=(jax.ShapeDtypeStruct((N_PAD, D), jnp.float32),
               jax.ShapeDtypeStruct((N_PAD, 8), jnp.float32)),
)

_tc2 = pl.pallas_call(
    _tc_layer2,
    grid=_GRID,
    in_specs=[
        pl.BlockSpec((NC, BM, D), lambda i: (0, i, 0)),
        pl.BlockSpec((BM, 8), lambda i: (i, 0)),
        pl.BlockSpec((BM, D), lambda i: (i, 0)),
        pl.BlockSpec((D, D), lambda i: (0, 0)),
        pl.BlockSpec((1, D), lambda i: (0, 0)),
        pl.BlockSpec((D, D), lambda i: (0, 0)),
    ],
    out_specs=pl.BlockSpec((BM, D), lambda i: (i, 0)),
    out_shape=jax.ShapeDtypeStruct((N_PAD, D), jnp.float32),
)


def kernel(matrix_nodes_features, edge_index, W1l, b1, W1r, W2l, b2, W2r):
  x = matrix_nodes_features.astype(jnp.float32)
  ei = edge_index.astype(jnp.int32)
  src = ei[0].reshape(NW, EPT)
  dst = ei[1].reshape(NW, EPT)
  srcp = jnp.concatenate(
      [src, jnp.zeros((NW, PAD_E + CHUNK), jnp.int32)],
      axis=1).reshape(NW, N_CH + 1, CHUNK)
  dstp = jnp.concatenate(
      [dst, jnp.full((NW, PAD_E + CHUNK), N, jnp.int32)],
      axis=1).reshape(NW, N_CH + 1, CHUNK)
  xp = jnp.concatenate([x, jnp.zeros((N_PAD - N, D), jnp.float32)], axis=0)
  b1r = b1.reshape(1, D).astype(jnp.float32)
  b2r = b2.reshape(1, D).astype(jnp.float32)

  cnt = _counts(dstp)
  p1 = _seg_sum(xp, srcp, dstp)
  h, inv8 = _tc1(p1, cnt, xp, W1l.astype(jnp.float32), b1r,
                 W1r.astype(jnp.float32))
  p2 = _seg_sum(h, srcp, dstp)
  out = _tc2(p2, inv8, h, W2l.astype(jnp.float32), b2r,
             W2r.astype(jnp.float32))
  return out[:N]
